# baseline (speedup 1.0000x reference)
import jax
import jax.numpy as jnp
from jax.experimental import pallas as pl


_N = 4 * 1024 * 1024
_C = 4
_CHUNK = 1048576


def _one_hot_body(o_ref):
    pid = pl.program_id(0)
    lid = pid // ((_N // _CHUNK) // _C)
    sub = jax.lax.broadcasted_iota(jnp.int32, (_C, _CHUNK), 0)
    o_ref[...] = (sub == lid).astype(jnp.int32)


def kernel(w0, w1, w2, w3, y):
    out = pl.pallas_call(
        _one_hot_body,
        grid=(_N // _CHUNK,),
        out_specs=pl.BlockSpec((_C, _CHUNK), lambda i: (0, i)),
        out_shape=jax.ShapeDtypeStruct((_C, _N), jnp.int32),
    )()
    return (out.T.astype(jnp.int64), y)


# speedup vs baseline: 1.0416x; 1.0416x over previous
import jax
import jax.numpy as jnp
from jax.experimental import pallas as pl


_N = 4 * 1024 * 1024
_C = 4
_CHUNK = 524288


def _one_hot_body(o_ref):
    pid = pl.program_id(0)
    lid = pid // ((_N // _CHUNK) // _C)
    sub = jax.lax.broadcasted_iota(jnp.int32, (_C, _CHUNK), 0)
    o_ref[...] = (sub == lid).astype(jnp.int32)


def kernel(w0, w1, w2, w3, y):
    out = pl.pallas_call(
        _one_hot_body,
        grid=(_N // _CHUNK,),
        out_specs=pl.BlockSpec((_C, _CHUNK), lambda i: (0, i)),
        out_shape=jax.ShapeDtypeStruct((_C, _N), jnp.int32),
    )()
    return (out.T.astype(jnp.int64), y)
